# baseline (speedup 1.0000x reference)
import jax
import jax.numpy as jnp
from jax import lax
from jax.experimental import pallas as pl
from jax.experimental.pallas import tpu as pltpu

BS = 128
NUM_EXPERTS = 64
TOP_K = 8
PROMPT_LEN = 64
D_MODEL = 1024
PB = 8


def _moe_kernel(score_ref, idx_ref, experts_ref, out_ref, w_ref):
    @pl.when(pl.program_id(0) == 0)
    def _compute_w():
        idx = idx_ref[...]
        e_ids = lax.broadcasted_iota(jnp.int32, (BS, NUM_EXPERTS), 1)
        counts = jnp.zeros((BS, NUM_EXPERTS), dtype=jnp.float32)
        for k in range(TOP_K):
            counts += (idx[:, k:k + 1] == e_ids).astype(jnp.float32)
        w_ref[...] = counts * score_ref[...]

    w = w_ref[...].astype(jnp.bfloat16)
    for p in range(PB):
        xb = experts_ref[:, p, :].astype(jnp.bfloat16)
        out_ref[:, p, :] = jnp.dot(w, xb, preferred_element_type=jnp.float32)


def kernel(selection_score, expert_indices, experts):
    idx = expert_indices.astype(jnp.int32)
    out = pl.pallas_call(
        _moe_kernel,
        grid=(PROMPT_LEN // PB,),
        in_specs=[
            pl.BlockSpec((BS, NUM_EXPERTS), lambda j: (0, 0)),
            pl.BlockSpec((BS, TOP_K), lambda j: (0, 0)),
            pl.BlockSpec((NUM_EXPERTS, PB, D_MODEL), lambda j: (0, j, 0)),
        ],
        out_specs=pl.BlockSpec((BS, PB, D_MODEL), lambda j: (0, j, 0)),
        out_shape=jax.ShapeDtypeStruct((BS, PROMPT_LEN, D_MODEL), jnp.float32),
        scratch_shapes=[pltpu.VMEM((BS, NUM_EXPERTS), jnp.float32)],
    )(selection_score, idx, experts)
    return out


# speedup vs baseline: 1.8010x; 1.8010x over previous
import jax
import jax.numpy as jnp
from jax import lax
from jax.experimental import pallas as pl
from jax.experimental.pallas import tpu as pltpu

BS = 128
NUM_EXPERTS = 64
TOP_K = 8
PROMPT_LEN = 64
D_MODEL = 1024
PB = 8


def _moe_kernel(score_ref, idx_ref, experts_ref, out_ref, w_ref):
    @pl.when(pl.program_id(0) == 0)
    def _compute_w():
        idx = idx_ref[...]
        e_ids = lax.broadcasted_iota(jnp.int32, (BS, NUM_EXPERTS), 1)
        counts = jnp.zeros((BS, NUM_EXPERTS), dtype=jnp.float32)
        for k in range(TOP_K):
            counts += (idx[:, k:k + 1] == e_ids).astype(jnp.float32)
        w_ref[...] = counts * score_ref[...]

    w = w_ref[...]
    xt = jnp.transpose(experts_ref[...], (1, 0, 2))
    outs = [jnp.dot(w, xt[p], preferred_element_type=jnp.float32)
            for p in range(PB)]
    out_ref[...] = jnp.transpose(jnp.stack(outs, axis=0), (1, 0, 2))


def kernel(selection_score, expert_indices, experts):
    idx = expert_indices.astype(jnp.int32)
    out = pl.pallas_call(
        _moe_kernel,
        grid=(PROMPT_LEN // PB,),
        in_specs=[
            pl.BlockSpec((BS, NUM_EXPERTS), lambda j: (0, 0)),
            pl.BlockSpec((BS, TOP_K), lambda j: (0, 0)),
            pl.BlockSpec((NUM_EXPERTS, PB, D_MODEL), lambda j: (0, j, 0)),
        ],
        out_specs=pl.BlockSpec((BS, PB, D_MODEL), lambda j: (0, j, 0)),
        out_shape=jax.ShapeDtypeStruct((BS, PROMPT_LEN, D_MODEL), jnp.float32),
        scratch_shapes=[pltpu.VMEM((BS, NUM_EXPERTS), jnp.float32)],
    )(selection_score, idx, experts)
    return out


# speedup vs baseline: 1.8700x; 1.0383x over previous
import jax
import jax.numpy as jnp
from jax import lax
from jax.experimental import pallas as pl
from jax.experimental.pallas import tpu as pltpu

BS = 128
NUM_EXPERTS = 64
TOP_K = 8
PROMPT_LEN = 64
D_MODEL = 1024
PB = 16


def _moe_kernel(score_ref, idx_ref, experts_ref, out_ref, w_ref):
    @pl.when(pl.program_id(0) == 0)
    def _compute_w():
        idx = idx_ref[...]
        e_ids = lax.broadcasted_iota(jnp.int32, (BS, NUM_EXPERTS), 1)
        counts = jnp.zeros((BS, NUM_EXPERTS), dtype=jnp.float32)
        for k in range(TOP_K):
            counts += (idx[:, k:k + 1] == e_ids).astype(jnp.float32)
        w_ref[...] = counts * score_ref[...]

    w = w_ref[...]
    xt = jnp.transpose(experts_ref[...], (1, 0, 2))
    outs = [jnp.dot(w, xt[p], preferred_element_type=jnp.float32)
            for p in range(PB)]
    out_ref[...] = jnp.transpose(jnp.stack(outs, axis=0), (1, 0, 2))


def kernel(selection_score, expert_indices, experts):
    idx = expert_indices.astype(jnp.int32)
    out = pl.pallas_call(
        _moe_kernel,
        grid=(PROMPT_LEN // PB,),
        in_specs=[
            pl.BlockSpec((BS, NUM_EXPERTS), lambda j: (0, 0)),
            pl.BlockSpec((BS, TOP_K), lambda j: (0, 0)),
            pl.BlockSpec((NUM_EXPERTS, PB, D_MODEL), lambda j: (0, j, 0)),
        ],
        out_specs=pl.BlockSpec((BS, PB, D_MODEL), lambda j: (0, j, 0)),
        out_shape=jax.ShapeDtypeStruct((BS, PROMPT_LEN, D_MODEL), jnp.float32),
        scratch_shapes=[pltpu.VMEM((BS, NUM_EXPERTS), jnp.float32)],
    )(selection_score, idx, experts)
    return out
